# Initial kernel scaffold; baseline (speedup 1.0000x reference)
import jax
import jax.numpy as jnp
from jax.experimental import pallas as pl


def kernel(x, edge_index, batch, edge_attr, W0, att_src0, att_dst0, We0, att_e0, b0, gamma0, beta0, W1, att_src1, att_dst1, We1, att_e1, b1, gamma1, beta1, W2, att_src2, att_dst2, We2, att_e2, b2, gamma2, beta2, fcW, fcb):
    raise NotImplementedError("write your pallas kernel here")


# speedup vs baseline: 20.6577x; 20.6577x over previous
import functools

import jax
import jax.numpy as jnp
from jax import lax
from jax.experimental import pallas as pl
from jax.experimental.pallas import tpu as pltpu
from jax.experimental.pallas import tpu_sc as plsc

N = 10000
H = 128
NG = 64
NC = 2
NS = 16
BE = 128
NP = 10240
ZB = NP // NS


def _round_up(x, m):
    return (x + m - 1) // m * m


def _ale_body(eat_ref, we0_ref, ae0_ref, we1_ref, ae1_ref, we2_ref, ae2_ref,
              out_ref):
    for l, (we_ref, ae_ref) in enumerate(
            ((we0_ref, ae0_ref), (we1_ref, ae1_ref), (we2_ref, ae2_ref))):
        vc = jnp.sum(we_ref[...] * ae_ref[...][None, :], axis=1)
        acc = eat_ref[0, :] * vc[0]
        for k in range(1, eat_ref.shape[0]):
            acc = acc + eat_ref[k, :] * vc[k]
        out_ref[l, :] = acc


def _tc_ale(eat, We0, ae0, We1, ae1, We2, ae2):
    E = eat.shape[1]
    return pl.pallas_call(
        _ale_body,
        out_shape=jax.ShapeDtypeStruct((3, E), jnp.float32),
    )(eat, We0, ae0, We1, ae1, We2, ae2)


def _prep0_body(x_ref, w_ref, as_ref, ad_ref, h_ref, al_ref):
    h = jnp.dot(x_ref[...], w_ref[...], preferred_element_type=jnp.float32)
    h_ref[...] = h
    al_ref[0, :] = jnp.sum(h * as_ref[...][None, :], axis=1)
    al_ref[1, :] = jnp.sum(h * ad_ref[...][None, :], axis=1)


def _tc_prep0(x, W, a_s, a_d):
    n = x.shape[0]
    return pl.pallas_call(
        _prep0_body,
        out_shape=[jax.ShapeDtypeStruct((n, H), jnp.float32),
                   jax.ShapeDtypeStruct((2, n), jnp.float32)],
    )(x, W, a_s, a_d)


def _prep_body(p_ref, dn_ref, b_ref, g_ref, be_ref, w_ref, as_ref, ad_ref,
               h_ref, al_ref):
    t = p_ref[0, :N, :] + p_ref[1, :N, :]
    dn = dn_ref[0, :N] + dn_ref[1, :N] + jnp.float32(1e-16)
    t = t / dn[:, None] + b_ref[...][None, :]
    mu = jnp.mean(t, axis=0)
    var = jnp.mean((t - mu[None, :]) ** 2, axis=0)
    t = (t - mu[None, :]) / jnp.sqrt(var + jnp.float32(1e-5))
    t = t * g_ref[...][None, :] + be_ref[...][None, :]
    t = jnp.maximum(t, jnp.float32(0.0))
    h = jnp.dot(t, w_ref[...], preferred_element_type=jnp.float32)
    h_ref[...] = h
    al_ref[0, :] = jnp.sum(h * as_ref[...][None, :], axis=1)
    al_ref[1, :] = jnp.sum(h * ad_ref[...][None, :], axis=1)


def _tc_prep(p, dn, b, g, be, W, a_s, a_d):
    return pl.pallas_call(
        _prep_body,
        out_shape=[jax.ShapeDtypeStruct((N, H), jnp.float32),
                   jax.ShapeDtypeStruct((2, N), jnp.float32)],
    )(p, dn, b, g, be, W, a_s, a_d)


def _final_body(p_ref, dn_ref, b_ref, g_ref, be_ref, batch_ref, fcw_ref,
                fcb_ref, out_ref):
    t = p_ref[0, :N, :] + p_ref[1, :N, :]
    dn = dn_ref[0, :N] + dn_ref[1, :N] + jnp.float32(1e-16)
    t = t / dn[:, None] + b_ref[...][None, :]
    mu = jnp.mean(t, axis=0)
    var = jnp.mean((t - mu[None, :]) ** 2, axis=0)
    t = (t - mu[None, :]) / jnp.sqrt(var + jnp.float32(1e-5))
    t = t * g_ref[...][None, :] + be_ref[...][None, :]
    t = jnp.maximum(t, jnp.float32(0.0))
    gid = lax.broadcasted_iota(jnp.int32, (NG, N), 0)
    oh = (gid == batch_ref[0, :][None, :]).astype(jnp.float32)
    pooled = jnp.dot(oh, t, preferred_element_type=jnp.float32)
    o = jnp.dot(pooled, fcw_ref[...],
                preferred_element_type=jnp.float32) + fcb_ref[...][None, :]
    m = jnp.max(o, axis=1, keepdims=True)
    e = jnp.exp(o - m)
    out_ref[...] = e / jnp.sum(e, axis=1, keepdims=True)


def _tc_final(p, dn, b, g, be, batch2, fcW, fcb):
    return pl.pallas_call(
        _final_body,
        out_shape=jax.ShapeDtypeStruct((NG, 2), jnp.float32),
    )(p, dn, b, g, be, batch2, fcW, fcb)


def _sc_layer_factory(num_blocks):
    bpt = num_blocks // (NC * NS)
    mesh = plsc.VectorSubcoreMesh(core_axis_name="c", subcore_axis_name="s")

    @functools.partial(
        pl.kernel,
        out_type=[jax.ShapeDtypeStruct((NC, NP, H), jnp.float32),
                  jax.ShapeDtypeStruct((NC, NP), jnp.float32)],
        mesh=mesh,
        compiler_params=pltpu.CompilerParams(needs_layout_passes=False),
        scratch_types=[
            pltpu.VMEM((N,), jnp.float32),
            pltpu.VMEM((N,), jnp.float32),
            pltpu.VMEM((BE,), jnp.int32),
            pltpu.VMEM((BE,), jnp.int32),
            pltpu.VMEM((BE,), jnp.float32),
            pltpu.VMEM((BE,), jnp.float32),
            pltpu.VMEM((BE, H), jnp.float32),
            pltpu.VMEM_SHARED((NP, H), jnp.float32),
            pltpu.VMEM_SHARED((NP,), jnp.float32),
            pltpu.SemaphoreType.DMA,
        ],
    )
    def sc_layer(h_h, al2_h, src_h, dst_h, ale_h, zrows_h, zn_h,
                 outp_h, denp_h,
                 als_v, ald_v, srcv, dstv, alev, exv, rows_v, out_s, den_s,
                 gsem):
        c = lax.axis_index("c")
        s = lax.axis_index("s")
        pltpu.sync_copy(al2_h.at[0], als_v)
        pltpu.sync_copy(al2_h.at[1], ald_v)
        pltpu.sync_copy(zrows_h.at[pl.ds(s * ZB, ZB)],
                        out_s.at[pl.ds(s * ZB, ZB)])
        pltpu.sync_copy(zn_h.at[pl.ds(s * ZB, ZB)],
                        den_s.at[pl.ds(s * ZB, ZB)])
        plsc.subcore_barrier()

        base = (c * NS + s) * bpt

        def block(blk, carry):
            off = (base + blk) * BE
            pltpu.sync_copy(src_h.at[pl.ds(off, BE)], srcv)
            pltpu.sync_copy(dst_h.at[pl.ds(off, BE)], dstv)
            pltpu.sync_copy(ale_h.at[pl.ds(off, BE)], alev)
            cp = pltpu.async_copy(h_h.at[srcv], rows_v, gsem)
            exs = []
            for g in range(BE // 16):
                sidx = srcv[pl.ds(g * 16, 16)]
                didx = dstv[pl.ds(g * 16, 16)]
                av = plsc.load_gather(als_v, [sidx])
                bv = plsc.load_gather(ald_v, [didx])
                lg = av + bv + alev[pl.ds(g * 16, 16)]
                lg = jnp.where(lg >= 0, lg, lg * jnp.float32(0.2))
                ex = jnp.exp(lg)
                exv[pl.ds(g * 16, 16)] = ex
                exs.append(ex)
            cp.wait()
            for g in range(BE // 16):
                ex = exs[g]
                for i in range(16):
                    e = g * 16 + i
                    sb = jnp.full((16,), ex[i], jnp.float32)
                    for k in range(H // 16):
                        rows_v[e, pl.ds(k * 16, 16)] = (
                            rows_v[e, pl.ds(k * 16, 16)] * sb)
            pltpu.sync_copy(rows_v, out_s.at[dstv], add=True)
            pltpu.sync_copy(exv, den_s.at[dstv], add=True)
            return carry

        lax.fori_loop(0, bpt, block, 0)
        plsc.subcore_barrier()
        pltpu.sync_copy(out_s.at[pl.ds(s * ZB, ZB)],
                        outp_h.at[c, pl.ds(s * ZB, ZB)])
        pltpu.sync_copy(den_s.at[pl.ds(s * ZB, ZB)],
                        denp_h.at[c, pl.ds(s * ZB, ZB)])

    return sc_layer


def kernel(x, edge_index, batch, edge_attr,
           W0, att_src0, att_dst0, We0, att_e0, b0, gamma0, beta0,
           W1, att_src1, att_dst1, We1, att_e1, b1, gamma1, beta1,
           W2, att_src2, att_dst2, We2, att_e2, b2, gamma2, beta2,
           fcW, fcb):
    E = edge_index.shape[1]
    EP = _round_up(E, BE * NC * NS)
    num_blocks = EP // BE
    pad = EP - E

    src = jnp.pad(edge_index[0], (0, pad))
    dst = jnp.pad(edge_index[1], (0, pad))

    eat = edge_attr.T
    ale3 = _tc_ale(eat, We0, att_e0, We1, att_e1, We2, att_e2)
    ale3 = jnp.pad(ale3, ((0, 0), (0, pad)), constant_values=-1e30)

    zrows = jnp.zeros((NP, H), jnp.float32)
    zn = jnp.zeros((NP,), jnp.float32)

    xp = jnp.pad(x, ((0, 0), (0, 1)))
    W0p = jnp.pad(W0, ((0, 1), (0, 0)))

    sc_layer = _sc_layer_factory(num_blocks)

    h, al2 = _tc_prep0(xp, W0p, att_src0, att_dst0)
    p, dn = sc_layer(h, al2, src, dst, ale3[0], zrows, zn)
    h, al2 = _tc_prep(p, dn, b0, gamma0, beta0, W1, att_src1, att_dst1)
    p, dn = sc_layer(h, al2, src, dst, ale3[1], zrows, zn)
    h, al2 = _tc_prep(p, dn, b1, gamma1, beta1, W2, att_src2, att_dst2)
    p, dn = sc_layer(h, al2, src, dst, ale3[2], zrows, zn)

    batch2 = batch.reshape(1, N)
    return _tc_final(p, dn, b2, gamma2, beta2, batch2, fcW, fcb)


# speedup vs baseline: 52.7658x; 2.5543x over previous
import functools

import jax
import jax.numpy as jnp
from jax import lax
from jax.experimental import pallas as pl
from jax.experimental.pallas import tpu as pltpu
from jax.experimental.pallas import tpu_sc as plsc

N = 10000
H = 128
NG = 64
NC = 2
NS = 16
BE = 128
CH = 8
NP = 10240
ZB = NP // NS


def _round_up(x, m):
    return (x + m - 1) // m * m


def _ale_body(eat_ref, we0_ref, ae0_ref, we1_ref, ae1_ref, we2_ref, ae2_ref,
              out_ref):
    for l, (we_ref, ae_ref) in enumerate(
            ((we0_ref, ae0_ref), (we1_ref, ae1_ref), (we2_ref, ae2_ref))):
        vc = jnp.sum(we_ref[...] * ae_ref[...][None, :], axis=1)
        acc = eat_ref[0, :] * vc[0]
        for k in range(1, eat_ref.shape[0]):
            acc = acc + eat_ref[k, :] * vc[k]
        out_ref[l, :] = acc


def _tc_ale(eat, We0, ae0, We1, ae1, We2, ae2):
    E = eat.shape[1]
    return pl.pallas_call(
        _ale_body,
        out_shape=jax.ShapeDtypeStruct((3, E), jnp.float32),
    )(eat, We0, ae0, We1, ae1, We2, ae2)


def _prep0_body(x_ref, w_ref, as_ref, ad_ref, h_ref, al_ref):
    h = jnp.dot(x_ref[...], w_ref[...], preferred_element_type=jnp.float32)
    h_ref[...] = h
    al_ref[0, :] = jnp.sum(h * as_ref[...][None, :], axis=1)
    al_ref[1, :] = jnp.sum(h * ad_ref[...][None, :], axis=1)


def _tc_prep0(x, W, a_s, a_d):
    n = x.shape[0]
    return pl.pallas_call(
        _prep0_body,
        out_shape=[jax.ShapeDtypeStruct((n, H), jnp.float32),
                   jax.ShapeDtypeStruct((2, n), jnp.float32)],
    )(x, W, a_s, a_d)


def _prep_body(p_ref, dn_ref, b_ref, g_ref, be_ref, w_ref, as_ref, ad_ref,
               h_ref, al_ref):
    t = p_ref[0, :N, :] + p_ref[1, :N, :]
    dn = dn_ref[0, 0, :N] + dn_ref[1, 0, :N] + jnp.float32(1e-16)
    t = t / dn[:, None] + b_ref[...][None, :]
    mu = jnp.mean(t, axis=0)
    var = jnp.mean((t - mu[None, :]) ** 2, axis=0)
    t = (t - mu[None, :]) / jnp.sqrt(var + jnp.float32(1e-5))
    t = t * g_ref[...][None, :] + be_ref[...][None, :]
    t = jnp.maximum(t, jnp.float32(0.0))
    h = jnp.dot(t, w_ref[...], preferred_element_type=jnp.float32)
    h_ref[...] = h
    al_ref[0, :] = jnp.sum(h * as_ref[...][None, :], axis=1)
    al_ref[1, :] = jnp.sum(h * ad_ref[...][None, :], axis=1)


def _tc_prep(p, dn, b, g, be, W, a_s, a_d):
    return pl.pallas_call(
        _prep_body,
        out_shape=[jax.ShapeDtypeStruct((N, H), jnp.float32),
                   jax.ShapeDtypeStruct((2, N), jnp.float32)],
    )(p, dn, b, g, be, W, a_s, a_d)


def _final_body(p_ref, dn_ref, b_ref, g_ref, be_ref, batch_ref, fcw_ref,
                fcb_ref, out_ref):
    t = p_ref[0, :N, :] + p_ref[1, :N, :]
    dn = dn_ref[0, 0, :N] + dn_ref[1, 0, :N] + jnp.float32(1e-16)
    t = t / dn[:, None] + b_ref[...][None, :]
    mu = jnp.mean(t, axis=0)
    var = jnp.mean((t - mu[None, :]) ** 2, axis=0)
    t = (t - mu[None, :]) / jnp.sqrt(var + jnp.float32(1e-5))
    t = t * g_ref[...][None, :] + be_ref[...][None, :]
    t = jnp.maximum(t, jnp.float32(0.0))
    gid = lax.broadcasted_iota(jnp.int32, (NG, N), 0)
    oh = (gid == batch_ref[0, :][None, :]).astype(jnp.float32)
    pooled = jnp.dot(oh, t, preferred_element_type=jnp.float32)
    o = jnp.dot(pooled, fcw_ref[...],
                preferred_element_type=jnp.float32) + fcb_ref[...][None, :]
    m = jnp.max(o, axis=1, keepdims=True)
    e = jnp.exp(o - m)
    out_ref[...] = e / jnp.sum(e, axis=1, keepdims=True)


def _tc_final(p, dn, b, g, be, batch2, fcW, fcb):
    return pl.pallas_call(
        _final_body,
        out_shape=jax.ShapeDtypeStruct((NG, 2), jnp.float32),
    )(p, dn, b, g, be, batch2, fcW, fcb)


def _sc_layer_factory(num_blocks):
    bpt = num_blocks // (NC * NS)
    nch = bpt // CH
    mesh = plsc.VectorSubcoreMesh(core_axis_name="c", subcore_axis_name="s")

    @functools.partial(
        pl.kernel,
        out_type=[jax.ShapeDtypeStruct((NC, NP, H), jnp.float32),
                  jax.ShapeDtypeStruct((NC, 1, NP), jnp.float32)],
        mesh=mesh,
        compiler_params=pltpu.CompilerParams(needs_layout_passes=False),
        scratch_types=[
            pltpu.VMEM((2, CH, BE), jnp.int32),
            pltpu.VMEM((2, CH, BE), jnp.int32),
            pltpu.VMEM((2, CH, BE), jnp.float32),
            pltpu.VMEM((2, 1, BE), jnp.float32),
            pltpu.VMEM((2, 1, BE), jnp.float32),
            pltpu.VMEM((2, 1, BE), jnp.float32),
            pltpu.VMEM((2, BE, H), jnp.float32),
            pltpu.VMEM_SHARED((NP, H), jnp.float32),
            pltpu.VMEM_SHARED((NP,), jnp.float32),
            pltpu.SemaphoreType.DMA,
            pltpu.SemaphoreType.DMA,
            pltpu.SemaphoreType.DMA((2,)),
        ],
    )
    def sc_layer(h_h, als_h, ald_h, src2_h, dst2_h, ale2_h, zrows_h, zn_h,
                 outp_h, denp_h,
                 esrc_v, edst_v, eale_v, alsg_v, aldg_v, exv, rows_v,
                 out_s, den_s, csem, gsem, ssem):
        c = lax.axis_index("c")
        s = lax.axis_index("s")
        pltpu.sync_copy(zrows_h.at[pl.ds(s * ZB, ZB)],
                        out_s.at[pl.ds(s * ZB, ZB)])
        pltpu.sync_copy(zn_h.at[pl.ds(s * ZB, ZB)],
                        den_s.at[pl.ds(s * ZB, ZB)])
        plsc.subcore_barrier()

        base = (c * NS + s) * bpt

        def chunk_fetch(ch, cbuf):
            gb0 = base + ch * CH
            pltpu.async_copy(src2_h.at[pl.ds(gb0, CH)], esrc_v.at[cbuf], csem)
            pltpu.async_copy(dst2_h.at[pl.ds(gb0, CH)], edst_v.at[cbuf], csem)
            pltpu.async_copy(ale2_h.at[pl.ds(gb0, CH)], eale_v.at[cbuf], csem)

        def chunk_wait(cbuf):
            pltpu.make_async_copy(src2_h.at[pl.ds(0, CH)], esrc_v.at[cbuf],
                                  csem).wait()
            pltpu.make_async_copy(dst2_h.at[pl.ds(0, CH)], edst_v.at[cbuf],
                                  csem).wait()
            pltpu.make_async_copy(ale2_h.at[pl.ds(0, CH)], eale_v.at[cbuf],
                                  csem).wait()

        def drain_scat(q):
            pltpu.make_async_copy(rows_v.at[q], out_s.at[edst_v.at[0, 0]],
                                  ssem.at[q]).wait()
            pltpu.make_async_copy(exv.at[q, 0], den_s.at[edst_v.at[0, 0]],
                                  ssem.at[q]).wait()

        def gather_rows(cbuf, b, q):
            pltpu.async_copy(h_h.at[esrc_v.at[cbuf, b]], rows_v.at[q], gsem)
            pltpu.async_copy(als_h.at[esrc_v.at[cbuf, b]], alsg_v.at[q, 0],
                             gsem)
            pltpu.async_copy(ald_h.at[edst_v.at[cbuf, b]], aldg_v.at[q, 0],
                             gsem)

        def gather_wait(cbuf, b, q):
            pltpu.make_async_copy(h_h.at[esrc_v.at[cbuf, b]], rows_v.at[q],
                                  gsem).wait()
            pltpu.make_async_copy(als_h.at[esrc_v.at[cbuf, b]],
                                  alsg_v.at[q, 0], gsem).wait()
            pltpu.make_async_copy(ald_h.at[edst_v.at[cbuf, b]],
                                  aldg_v.at[q, 0], gsem).wait()

        chunk_fetch(0, 0)

        def chunk_body(ch, carry):
            cbuf = lax.rem(ch, 2)
            chunk_wait(cbuf)

            @pl.when(ch + 1 < nch)
            def _():
                chunk_fetch(ch + 1, 1 - cbuf)

            @pl.when(ch >= 1)
            def _():
                drain_scat(0)
            gather_rows(cbuf, 0, 0)

            def block(b, carry2):
                p = lax.rem(b, 2)
                gb = ch * CH + b

                @pl.when(b + 1 < CH)
                def _():
                    @pl.when(gb >= 1)
                    def _():
                        drain_scat(1 - p)
                    gather_rows(cbuf, b + 1, 1 - p)

                gather_wait(cbuf, b, p)
                exs = []
                for g in range(BE // 16):
                    av = alsg_v[p, 0, pl.ds(g * 16, 16)]
                    bv = aldg_v[p, 0, pl.ds(g * 16, 16)]
                    lg = av + bv + eale_v[cbuf, b, pl.ds(g * 16, 16)]
                    lg = jnp.where(lg >= 0, lg, lg * jnp.float32(0.2))
                    ex = jnp.exp(lg)
                    exv[p, 0, pl.ds(g * 16, 16)] = ex
                    exs.append(ex)
                for g in range(BE // 16):
                    ex = exs[g]
                    for i in range(16):
                        e = g * 16 + i
                        sb = jnp.full((16,), ex[i], jnp.float32)
                        for k in range(H // 16):
                            rows_v[p, e, pl.ds(k * 16, 16)] = (
                                rows_v[p, e, pl.ds(k * 16, 16)] * sb)
                pltpu.async_copy(rows_v.at[p], out_s.at[edst_v.at[cbuf, b]],
                                 ssem.at[p], add=True)
                pltpu.async_copy(exv.at[p, 0], den_s.at[edst_v.at[cbuf, b]],
                                 ssem.at[p], add=True)
                return carry2

            lax.fori_loop(0, CH, block, 0)
            return carry

        lax.fori_loop(0, nch, chunk_body, 0)
        drain_scat(0)
        drain_scat(1)
        plsc.subcore_barrier()
        pltpu.sync_copy(out_s.at[pl.ds(s * ZB, ZB)],
                        outp_h.at[c, pl.ds(s * ZB, ZB)])
        pltpu.sync_copy(den_s.at[pl.ds(s * ZB, ZB)],
                        denp_h.at[c, 0, pl.ds(s * ZB, ZB)])

    return sc_layer


def kernel(x, edge_index, batch, edge_attr,
           W0, att_src0, att_dst0, We0, att_e0, b0, gamma0, beta0,
           W1, att_src1, att_dst1, We1, att_e1, b1, gamma1, beta1,
           W2, att_src2, att_dst2, We2, att_e2, b2, gamma2, beta2,
           fcW, fcb):
    E = edge_index.shape[1]
    EP = _round_up(E, BE * NC * NS * CH)
    num_blocks = EP // BE
    pad = EP - E

    spread = (jnp.arange(pad, dtype=jnp.int32) % jnp.int32(N))
    src = jnp.concatenate([edge_index[0], spread]).reshape(num_blocks, BE)
    dst = jnp.concatenate([edge_index[1], spread]).reshape(num_blocks, BE)

    eat = edge_attr.T
    ale3 = _tc_ale(eat, We0, att_e0, We1, att_e1, We2, att_e2)
    ale3 = jnp.pad(ale3, ((0, 0), (0, pad)),
                   constant_values=-1e30).reshape(3, num_blocks, BE)

    zrows = jnp.zeros((NP, H), jnp.float32)
    zn = jnp.zeros((NP,), jnp.float32)

    xp = jnp.pad(x, ((0, 0), (0, 1)))
    W0p = jnp.pad(W0, ((0, 1), (0, 0)))

    sc_layer = _sc_layer_factory(num_blocks)

    h, al2 = _tc_prep0(xp, W0p, att_src0, att_dst0)
    p, dn = sc_layer(h, al2[0], al2[1], src, dst, ale3[0], zrows, zn)
    h, al2 = _tc_prep(p, dn, b0, gamma0, beta0, W1, att_src1, att_dst1)
    p, dn = sc_layer(h, al2[0], al2[1], src, dst, ale3[1], zrows, zn)
    h, al2 = _tc_prep(p, dn, b1, gamma1, beta1, W2, att_src2, att_dst2)
    p, dn = sc_layer(h, al2[0], al2[1], src, dst, ale3[2], zrows, zn)

    batch2 = batch.reshape(1, N)
    return _tc_final(p, dn, b2, gamma2, beta2, batch2, fcW, fcb)
